# baseline (speedup 1.0000x reference)
import jax
import jax.numpy as jnp
from jax.experimental import pallas as pl
from jax.experimental.pallas import tpu as pltpu

DIM = 512
BN = 4096


def _pool_kernel(b_ref, x0_ref, x1_ref, w_ref, out_ref, acc_ref, ssum_ref):
    j = pl.program_id(1)
    nj = pl.num_programs(1)

    @pl.when(j == 0)
    def _init():
        acc_ref[...] = jnp.zeros_like(acc_ref)
        ssum_ref[0, 0] = 0.0

    ba = b_ref[0, 0]
    bg = b_ref[0, 1]
    part = jnp.zeros_like(acc_ref)
    ssum = 0.0
    for x_ref in (x0_ref, x1_ref):
        x = x_ref[0]
        proj = jnp.dot(x, w_ref[...], preferred_element_type=jnp.float32)
        a = jnp.tanh(proj[:, 0:1] + ba)
        g = jax.nn.sigmoid(proj[:, 1:2] + bg)
        e = jnp.exp(a * g)
        part = part + jnp.sum(e * x, axis=0, keepdims=True)
        ssum = ssum + jnp.sum(e)
    acc_ref[...] += part
    ssum_ref[0, 0] += ssum

    @pl.when(j == nj - 1)
    def _finish():
        out_ref[0] = acc_ref[...] / ssum_ref[0, 0]


def kernel(x, W_a, b_a, W_g, b_g):
    B, N, D = x.shape
    w = jnp.zeros((D, 128), jnp.float32).at[:, 0].set(W_a[:, 0]).at[:, 1].set(W_g[:, 0])
    biases = jnp.stack([b_a[0], b_g[0]]).reshape(1, 2)

    nj = N // (2 * BN)
    out = pl.pallas_call(
        _pool_kernel,
        grid=(B, nj),
        in_specs=[
            pl.BlockSpec(memory_space=pltpu.SMEM),
            pl.BlockSpec((1, BN, D), lambda b, j: (b, 2 * j, 0)),
            pl.BlockSpec((1, BN, D), lambda b, j: (b, 2 * j + 1, 0)),
            pl.BlockSpec((D, 128), lambda b, j: (0, 0)),
        ],
        out_specs=pl.BlockSpec((1, 1, D), lambda b, j: (b, 0, 0)),
        out_shape=jax.ShapeDtypeStruct((B, 1, D), jnp.float32),
        scratch_shapes=[
            pltpu.VMEM((1, D), jnp.float32),
            pltpu.SMEM((1, 1), jnp.float32),
        ],
        compiler_params=pltpu.CompilerParams(
            dimension_semantics=("parallel", "arbitrary"),
        ),
    )(biases, x, x, w)
    return out
